# baseline (speedup 1.0000x reference)
import functools

import jax
import jax.numpy as jnp
from jax.experimental import pallas as pl
from jax.experimental.pallas import tpu as pltpu

B, N = 32, 512
TI = 256


def _fused_kernel(xs_ref, ys_ref, x2_ref, y2_ref, xt_ref, yt_ref,
                  tour_ref, tnext_ref, wrow_ref, p_ref,
                  yp_ref, xev_ref, lsum_ref):
    b = pl.program_id(0)
    r = pl.program_id(1)

    wc0 = p_ref[0]
    wc1 = p_ref[1]
    emb1 = p_ref[2]
    we0 = p_ref[3]
    we1 = p_ref[4]
    be0 = p_ref[5]
    be1 = p_ref[6]

    x = xs_ref[0, 0, :]
    y = ys_ref[0, 0, :]
    x2 = x2_ref[0, 0, :]
    y2 = y2_ref[0, 0, :]
    xt = xt_ref[0, 0, :]
    yt = yt_ref[0, 0, :]
    we_row = wrow_ref[0, 0, :]
    be_row = wrow_ref[0, 1, :]

    dx = xt[:, None] - x[None, :]
    dy = yt[:, None] - y[None, :]
    d = jnp.sqrt(dx * dx + dy * dy)
    xev_ref[0, :, :] = d

    st = xt * wc0 + yt * wc1 + emb1
    s2 = x2 * wc0 + y2 * wc1 + emb1
    dx2 = xt[:, None] - x2[None, :]
    dy2 = yt[:, None] - y2[None, :]
    d2 = jnp.sqrt(dx2 * dx2 + dy2 * dy2)
    yp_ref[0, :, :] = d2 * we_row[None, :] + be_row[None, :] \
        + st[:, None] + s2[None, :]

    tour = tour_ref[0, 0, :]
    tnext = tnext_ref[0, 0, :]
    col_full = jax.lax.broadcasted_iota(jnp.int32, (N, N), 1)
    col_tile = jax.lax.broadcasted_iota(jnp.int32, (N, TI), 1) + r * TI
    a_full = (tour[:, None] == col_full).astype(jnp.bfloat16)
    bn_full = (tnext[:, None] == col_full).astype(jnp.bfloat16)
    a_tile = (tour[:, None] == col_tile).astype(jnp.bfloat16)
    bn_tile = (tnext[:, None] == col_tile).astype(jnp.bfloat16)
    dn = (((0,), (0,)), ((), ()))
    m_fwd = jax.lax.dot_general(a_tile, bn_full, dn,
                                preferred_element_type=jnp.float32)
    m_bwd = jax.lax.dot_general(bn_tile, a_full, dn,
                                preferred_element_type=jnp.float32)
    mask = (m_fwd + m_bwd) > 0.0

    a0 = d * we0 + be0
    a1 = d * we1 + be1
    mx = jnp.maximum(a0, a1)
    lse = mx + jnp.log1p(jnp.exp(jnp.minimum(a0, a1) - mx))
    sel = jnp.where(mask, a1, a0) - lse

    @pl.when((b == 0) & (r == 0))
    def _():
        lsum_ref[0, 0] = 0.0

    lsum_ref[0, 0] += jnp.sum(sel)


@functools.partial(jax.jit, static_argnames=("interpret",))
def kernel(x_nodes_coord, y_tour, w_coord, emb, w_e, b_e, interpret=False):
    xs = x_nodes_coord[:, :, 0].reshape(B, 1, N)
    ys = x_nodes_coord[:, :, 1].reshape(B, 1, N)
    x2 = jnp.broadcast_to(x_nodes_coord[:, :, :1], (B, N, 2)).reshape(B, 1, 2 * N)
    y2 = jnp.broadcast_to(x_nodes_coord[:, :, 1:], (B, N, 2)).reshape(B, 1, 2 * N)
    tour = y_tour.reshape(B, 1, N)
    tnext = jnp.roll(y_tour, -1, axis=-1).reshape(B, 1, N)
    wrow = jnp.stack([jnp.tile(w_e, N), jnp.tile(b_e, N)]).reshape(1, 2, 2 * N)
    params = jnp.stack([w_coord[0], w_coord[1], emb[1],
                        w_e[0], w_e[1], b_e[0], b_e[1]])

    full_spec = pl.BlockSpec((1, 1, N), lambda b, r: (b, 0, 0))
    dbl_spec = pl.BlockSpec((1, 1, 2 * N), lambda b, r: (b, 0, 0))
    tile_spec = pl.BlockSpec((1, 1, TI), lambda b, r: (b, 0, r))
    yp, xev, lsum = pl.pallas_call(
        _fused_kernel,
        grid=(B, N // TI),
        in_specs=[full_spec, full_spec, dbl_spec, dbl_spec,
                  tile_spec, tile_spec, full_spec, full_spec,
                  pl.BlockSpec((1, 2, 2 * N), lambda b, r: (0, 0, 0)),
                  pl.BlockSpec(memory_space=pltpu.SMEM)],
        out_specs=[
            pl.BlockSpec((1, TI, 2 * N), lambda b, r: (b, r, 0)),
            pl.BlockSpec((1, TI, N), lambda b, r: (b, r, 0)),
            pl.BlockSpec((1, 1), lambda b, r: (0, 0), memory_space=pltpu.SMEM),
        ],
        out_shape=[
            jax.ShapeDtypeStruct((B, N, 2 * N), jnp.float32),
            jax.ShapeDtypeStruct((B, N, N), jnp.float32),
            jax.ShapeDtypeStruct((1, 1), jnp.float32),
        ],
        interpret=interpret,
    )(xs, ys, x2, y2, xs, ys, tour, tnext, wrow, params)

    y_preds = yp.reshape(B, N, N, 2)
    loss = -lsum[0, 0] / jnp.float32(B * N * N)
    return (y_preds, loss, xev)


# speedup vs baseline: 1.1198x; 1.1198x over previous
import functools

import jax
import jax.numpy as jnp
from jax.experimental import pallas as pl
from jax.experimental.pallas import tpu as pltpu

B, N = 32, 512
TI = 256


def _fused_kernel(xs_ref, ys_ref, xt_ref, yt_ref, tour_ref, tnext_ref,
                  wrow_ref, p0_ref, p_ref, yp_ref, xev_ref, lsum_ref):
    b = pl.program_id(0)
    r = pl.program_id(1)

    wc0 = p_ref[0]
    wc1 = p_ref[1]
    emb1 = p_ref[2]
    we0 = p_ref[3]
    we1 = p_ref[4]
    be0 = p_ref[5]
    be1 = p_ref[6]

    x = xs_ref[0, 0, :]
    y = ys_ref[0, 0, :]
    xt = xt_ref[0, 0, :]
    yt = yt_ref[0, 0, :]
    we_row = wrow_ref[0, 0, :]
    be_row = wrow_ref[0, 1, :]
    p0m = p0_ref[0]

    dx = xt[:, None] - x[None, :]
    dy = yt[:, None] - y[None, :]
    d = jnp.sqrt(dx * dx + dy * dy)
    xev_ref[0, :, :] = d

    st = xt * wc0 + yt * wc1 + emb1
    s = x * wc0 + y * wc1 + emb1
    s2 = jnp.dot(s[None, :], p0m, preferred_element_type=jnp.float32)[0]
    d2 = jnp.dot(d, p0m, preferred_element_type=jnp.float32)
    yp_ref[0, :, :] = d2 * we_row[None, :] + (be_row + s2)[None, :] \
        + st[:, None]

    tour = tour_ref[0, 0, :]
    tnext = tnext_ref[0, 0, :]
    col_full = jax.lax.broadcasted_iota(jnp.int32, (N, N), 1)
    col_tile = jax.lax.broadcasted_iota(jnp.int32, (N, TI), 1) + r * TI
    a_full = (tour[:, None] == col_full).astype(jnp.bfloat16)
    bn_full = (tnext[:, None] == col_full).astype(jnp.bfloat16)
    a_tile = (tour[:, None] == col_tile).astype(jnp.bfloat16)
    bn_tile = (tnext[:, None] == col_tile).astype(jnp.bfloat16)
    dn = (((0,), (0,)), ((), ()))
    m_fwd = jax.lax.dot_general(a_tile, bn_full, dn,
                                preferred_element_type=jnp.float32)
    m_bwd = jax.lax.dot_general(bn_tile, a_full, dn,
                                preferred_element_type=jnp.float32)
    mask = (m_fwd + m_bwd) > 0.0

    a0 = d * we0 + be0
    a1 = d * we1 + be1
    mx = jnp.maximum(a0, a1)
    lse = mx + jnp.log1p(jnp.exp(jnp.minimum(a0, a1) - mx))
    sel = jnp.where(mask, a1, a0) - lse

    @pl.when((b == 0) & (r == 0))
    def _():
        lsum_ref[0, 0] = 0.0

    lsum_ref[0, 0] += jnp.sum(sel)


@functools.partial(jax.jit, static_argnames=("interpret",))
def kernel(x_nodes_coord, y_tour, w_coord, emb, w_e, b_e, interpret=False):
    xs = x_nodes_coord[:, :, 0].reshape(B, 1, N)
    ys = x_nodes_coord[:, :, 1].reshape(B, 1, N)
    tour = y_tour.reshape(B, 1, N)
    tnext = jnp.roll(y_tour, -1, axis=-1).reshape(B, 1, N)
    wrow = jnp.stack([jnp.tile(w_e, N), jnp.tile(b_e, N)]).reshape(1, 2, 2 * N)
    p0 = (jnp.arange(2 * N, dtype=jnp.int32)[None, :] // 2
          == jnp.arange(N, dtype=jnp.int32)[:, None])
    p0 = p0.astype(jnp.float32).reshape(1, N, 2 * N)
    params = jnp.stack([w_coord[0], w_coord[1], emb[1],
                        w_e[0], w_e[1], b_e[0], b_e[1]])

    full_spec = pl.BlockSpec((1, 1, N), lambda b, r: (b, 0, 0))
    tile_spec = pl.BlockSpec((1, 1, TI), lambda b, r: (b, 0, r))
    yp, xev, lsum = pl.pallas_call(
        _fused_kernel,
        grid=(B, N // TI),
        in_specs=[full_spec, full_spec, tile_spec, tile_spec,
                  full_spec, full_spec,
                  pl.BlockSpec((1, 2, 2 * N), lambda b, r: (0, 0, 0)),
                  pl.BlockSpec((1, N, 2 * N), lambda b, r: (0, 0, 0)),
                  pl.BlockSpec(memory_space=pltpu.SMEM)],
        out_specs=[
            pl.BlockSpec((1, TI, 2 * N), lambda b, r: (b, r, 0)),
            pl.BlockSpec((1, TI, N), lambda b, r: (b, r, 0)),
            pl.BlockSpec((1, 1), lambda b, r: (0, 0), memory_space=pltpu.SMEM),
        ],
        out_shape=[
            jax.ShapeDtypeStruct((B, N, 2 * N), jnp.float32),
            jax.ShapeDtypeStruct((B, N, N), jnp.float32),
            jax.ShapeDtypeStruct((1, 1), jnp.float32),
        ],
        interpret=interpret,
    )(xs, ys, xs, ys, tour, tnext, wrow, p0, params)

    y_preds = yp.reshape(B, N, N, 2)
    loss = -lsum[0, 0] / jnp.float32(B * N * N)
    return (y_preds, loss, xev)
